# baseline (speedup 1.0000x reference)
import functools

import jax
import jax.numpy as jnp
from jax import lax
from jax.experimental import pallas as pl
from jax.experimental.pallas import tpu as pltpu
from jax.experimental.pallas import tpu_sc as plsc

N_NODES = 10000
N_EDGES = 320000
D_FEAT = 128
N_CLASSES = 47

NC = 2
NS = 16
NW = NC * NS
CHUNK = 128
NCH = 80
EPAD = NW * NCH * CHUNK
D0 = 144
D1 = 48
NPAD = 10016
ROWS_PER_TILE = NPAD // NS

BLK = 1000


def _make_sc_scatter(d, nbuf, g, stage_table=False):
    mesh = plsc.VectorSubcoreMesh(core_axis_name="c", subcore_axis_name="s")
    scratch = [
        pltpu.VMEM((g, CHUNK), jnp.int32),
        pltpu.VMEM((g, CHUNK), jnp.int32),
        pltpu.VMEM((nbuf, CHUNK, d), jnp.float32),
        pltpu.VMEM_SHARED((NPAD, d), jnp.float32),
        [pltpu.SemaphoreType.DMA] * nbuf,
    ]
    if stage_table:
        scratch.append(pltpu.VMEM_SHARED((N_NODES, d), jnp.float32))

    @functools.partial(
        pl.kernel,
        out_type=jax.ShapeDtypeStruct((NC, NPAD, d), jnp.float32),
        mesh=mesh,
        scratch_types=scratch,
        compiler_params=pltpu.CompilerParams(use_tc_tiling_on_sc=False),
    )
    def sc_scatter(y_hbm, src_hbm, dst_hbm, z_hbm, out_hbm,
                   srcv, dstv, rows, acc, sems, *maybe_tab):
        cid = lax.axis_index("c")
        sid = lax.axis_index("s")
        r0 = sid * ROWS_PER_TILE
        pltpu.sync_copy(z_hbm.at[pl.ds(r0, ROWS_PER_TILE)],
                        acc.at[pl.ds(r0, ROWS_PER_TILE)])
        if stage_table:
            tab = maybe_tab[0]
            tr = N_NODES // NS
            pltpu.sync_copy(y_hbm.at[pl.ds(sid * tr, tr)],
                            tab.at[pl.ds(sid * tr, tr)])
            src_tab = tab
        else:
            src_tab = y_hbm
        plsc.subcore_barrier()

        def group(gi, carry):
            pltpu.sync_copy(src_hbm.at[sid, pl.ds(cid * NCH + gi * g, g)], srcv)
            pltpu.sync_copy(dst_hbm.at[sid, pl.ds(cid * NCH + gi * g, g)], dstv)
            for b in range(nbuf):
                pltpu.async_copy(src_tab.at[srcv.at[b]], rows.at[b], sems[b])

            def body(t, carry2):
                for b in range(nbuf):
                    j = t * nbuf + b
                    pltpu.make_async_copy(
                        src_tab.at[srcv.at[j]], rows.at[b], sems[b]).wait()
                    pltpu.sync_copy(rows.at[b], acc.at[dstv.at[j]], add=True)

                    @pl.when(j + nbuf < g)
                    def _():
                        pltpu.async_copy(
                            src_tab.at[srcv.at[j + nbuf]], rows.at[b], sems[b])
                return carry2

            return lax.fori_loop(0, g // nbuf, body, carry)

        lax.fori_loop(0, NCH // g, group, 0)
        plsc.subcore_barrier()
        pltpu.sync_copy(acc.at[pl.ds(r0, ROWS_PER_TILE)],
                        out_hbm.at[cid, pl.ds(r0, ROWS_PER_TILE)])

    return sc_scatter


_sc_scatter_d1 = _make_sc_scatter(D1, nbuf=4, g=80, stage_table=True)

DSP = 64
DDEG = 16
NCH2 = 160


def _make_sc_scatter_split(nbuf, g):
    mesh = plsc.VectorSubcoreMesh(core_axis_name="c", subcore_axis_name="s")

    @functools.partial(
        pl.kernel,
        out_type=(jax.ShapeDtypeStruct((NC, NPAD, DSP), jnp.float32),
                  jax.ShapeDtypeStruct((NC, NPAD, DDEG), jnp.float32)),
        mesh=mesh,
        scratch_types=[
            pltpu.VMEM((g, CHUNK), jnp.int32),
            pltpu.VMEM((g, CHUNK), jnp.int32),
            pltpu.VMEM((nbuf, CHUNK, DSP), jnp.float32),
            pltpu.VMEM((CHUNK, DDEG), jnp.float32),
            pltpu.VMEM_SHARED((NPAD, DSP), jnp.float32),
            pltpu.VMEM_SHARED((NPAD, DDEG), jnp.float32),
            pltpu.VMEM_SHARED((N_NODES, DSP), jnp.float32),
            [pltpu.SemaphoreType.DMA] * nbuf,
        ],
        compiler_params=pltpu.CompilerParams(use_tc_tiling_on_sc=False),
    )
    def sc_scatter(ya_hbm, yb_hbm, src_hbm, dst_hbm, z_hbm, zd_hbm, ones_hbm,
                   out_hbm, deg_hbm, srcv, dstv, rows, onesv, acc, dacc, tab,
                   sems):
        cid = lax.axis_index("c")
        sid = lax.axis_index("s")
        r0 = sid * ROWS_PER_TILE
        pltpu.sync_copy(z_hbm.at[pl.ds(r0, ROWS_PER_TILE)],
                        acc.at[pl.ds(r0, ROWS_PER_TILE)])
        pltpu.sync_copy(zd_hbm.at[pl.ds(r0, ROWS_PER_TILE)],
                        dacc.at[pl.ds(r0, ROWS_PER_TILE)])
        pltpu.sync_copy(ones_hbm, onesv)
        tr = N_NODES // NS

        @pl.when(cid == 0)
        def _():
            pltpu.sync_copy(ya_hbm.at[pl.ds(sid * tr, tr)],
                            tab.at[pl.ds(sid * tr, tr)])

        @pl.when(cid == 1)
        def _():
            pltpu.sync_copy(yb_hbm.at[pl.ds(sid * tr, tr)],
                            tab.at[pl.ds(sid * tr, tr)])

        plsc.subcore_barrier()

        def group(gi, carry):
            pltpu.sync_copy(src_hbm.at[sid, pl.ds(gi * g, g)], srcv)
            pltpu.sync_copy(dst_hbm.at[sid, pl.ds(gi * g, g)], dstv)
            for b in range(nbuf):
                pltpu.async_copy(tab.at[srcv.at[b]], rows.at[b], sems[b])

            @pl.when(cid == 0)
            def _():
                for jj in range(g // 2):
                    pltpu.sync_copy(onesv, dacc.at[dstv.at[jj]], add=True)

            @pl.when(cid == 1)
            def _():
                for jj in range(g // 2, g):
                    pltpu.sync_copy(onesv, dacc.at[dstv.at[jj]], add=True)

            def body(t, carry2):
                for b in range(nbuf):
                    j = t * nbuf + b
                    pltpu.make_async_copy(
                        tab.at[srcv.at[j]], rows.at[b], sems[b]).wait()
                    pltpu.sync_copy(rows.at[b], acc.at[dstv.at[j]], add=True)

                    @pl.when(j + nbuf < g)
                    def _():
                        pltpu.async_copy(
                            tab.at[srcv.at[j + nbuf]], rows.at[b], sems[b])
                return carry2

            return lax.fori_loop(0, g // nbuf, body, carry)

        lax.fori_loop(0, NCH2 // g, group, 0)
        plsc.subcore_barrier()
        pltpu.sync_copy(acc.at[pl.ds(r0, ROWS_PER_TILE)],
                        out_hbm.at[cid, pl.ds(r0, ROWS_PER_TILE)])
        pltpu.sync_copy(dacc.at[pl.ds(r0, ROWS_PER_TILE)],
                        deg_hbm.at[cid, pl.ds(r0, ROWS_PER_TILE)])

    return sc_scatter


_sc_scatter_l0 = _make_sc_scatter_split(nbuf=4, g=16)


def _mm0_body(x_ref, w0a_ref, w0b_ref, ws_ref, b0_ref,
              ya_ref, yb_ref, s0_ref):
    xb = x_ref[...]
    ya_ref[...] = jnp.dot(xb, w0a_ref[...], preferred_element_type=jnp.float32)
    yb_ref[...] = jnp.dot(xb, w0b_ref[...], preferred_element_type=jnp.float32)
    s0_ref[...] = jnp.dot(xb, ws_ref[...],
                          preferred_element_type=jnp.float32) + b0_ref[...]


def _mid_body(acc_ref, deg_ref, s0_ref, w1_ref, ws1_ref, b1_ref,
              y1_ref, z1_ref, dinv_ref):
    agg = jnp.concatenate([acc_ref[0], acc_ref[1]], axis=1)
    deg = deg_ref[0][:, 0:1] + deg_ref[1][:, 0:1]
    dinv = 1.0 / jnp.maximum(deg, 1.0)
    h = jnp.maximum(agg * dinv + s0_ref[...], 0.0)
    y1_ref[...] = jnp.dot(h, w1_ref[...], preferred_element_type=jnp.float32)
    z1_ref[...] = jnp.dot(h, ws1_ref[...],
                          preferred_element_type=jnp.float32) + b1_ref[...]
    dinv_ref[...] = dinv


def _fin_body(acc_ref, dinv_ref, z1_ref, out_ref):
    a = acc_ref[0] + acc_ref[1]
    out_ref[...] = a[:, :N_CLASSES] * dinv_ref[...] + z1_ref[...]


def kernel(x, edge_index, W_neigh_0, W_self_0, b_0, W_neigh_1, W_self_1, b_1):
    src = edge_index[0].astype(jnp.int32)
    dst = edge_index[1].astype(jnp.int32)
    epw0 = N_EDGES // NS
    ppw0 = NCH2 * CHUNK - epw0
    pad_src0 = jnp.zeros((NS, ppw0), jnp.int32)
    pad_dst0 = jnp.broadcast_to(
        N_NODES + (jnp.arange(ppw0, dtype=jnp.int32) % (NPAD - N_NODES)),
        (NS, ppw0))
    src0_p = jnp.concatenate(
        [src.reshape(NS, epw0), pad_src0], axis=1).reshape(NS, NCH2, CHUNK)
    dst0_p = jnp.concatenate(
        [dst.reshape(NS, epw0), pad_dst0], axis=1).reshape(NS, NCH2, CHUNK)
    zeros_sp = jnp.zeros((NPAD, DSP), jnp.float32)
    zeros1 = jnp.zeros((NPAD, D1), jnp.float32)

    w0a = W_neigh_0[:, :DSP]
    w0b = W_neigh_0[:, DSP:]
    w1p = jnp.pad(W_neigh_1, ((0, 0), (0, D1 - N_CLASSES)))

    grid = N_NODES // BLK
    y0a, y0b, s0 = pl.pallas_call(
        _mm0_body,
        grid=(grid,),
        in_specs=[
            pl.BlockSpec((BLK, D_FEAT), lambda i: (i, 0)),
            pl.BlockSpec((D_FEAT, DSP), lambda i: (0, 0)),
            pl.BlockSpec((D_FEAT, DSP), lambda i: (0, 0)),
            pl.BlockSpec((D_FEAT, D_FEAT), lambda i: (0, 0)),
            pl.BlockSpec((1, D_FEAT), lambda i: (0, 0)),
        ],
        out_specs=[
            pl.BlockSpec((BLK, DSP), lambda i: (i, 0)),
            pl.BlockSpec((BLK, DSP), lambda i: (i, 0)),
            pl.BlockSpec((BLK, D_FEAT), lambda i: (i, 0)),
        ],
        out_shape=[
            jax.ShapeDtypeStruct((N_NODES, DSP), jnp.float32),
            jax.ShapeDtypeStruct((N_NODES, DSP), jnp.float32),
            jax.ShapeDtypeStruct((N_NODES, D_FEAT), jnp.float32),
        ],
    )(x, w0a, w0b, W_self_0, b_0[None, :])

    zeros_d = jnp.zeros((NPAD, DDEG), jnp.float32)
    ones_r = jnp.ones((CHUNK, DDEG), jnp.float32)
    acc0, degs = _sc_scatter_l0(y0a, y0b, src0_p, dst0_p, zeros_sp,
                                zeros_d, ones_r)

    y1p, z1, dinv = pl.pallas_call(
        _mid_body,
        grid=(grid,),
        in_specs=[
            pl.BlockSpec((NC, BLK, DSP), lambda i: (0, i, 0)),
            pl.BlockSpec((NC, BLK, DDEG), lambda i: (0, i, 0)),
            pl.BlockSpec((BLK, D_FEAT), lambda i: (i, 0)),
            pl.BlockSpec((D_FEAT, D1), lambda i: (0, 0)),
            pl.BlockSpec((D_FEAT, N_CLASSES), lambda i: (0, 0)),
            pl.BlockSpec((1, N_CLASSES), lambda i: (0, 0)),
        ],
        out_specs=[
            pl.BlockSpec((BLK, D1), lambda i: (i, 0)),
            pl.BlockSpec((BLK, N_CLASSES), lambda i: (i, 0)),
            pl.BlockSpec((BLK, 1), lambda i: (i, 0)),
        ],
        out_shape=[
            jax.ShapeDtypeStruct((N_NODES, D1), jnp.float32),
            jax.ShapeDtypeStruct((N_NODES, N_CLASSES), jnp.float32),
            jax.ShapeDtypeStruct((N_NODES, 1), jnp.float32),
        ],
    )(acc0, degs, s0, w1p, W_self_1, b_1[None, :])

    acc1 = _sc_scatter_d1(y1p, src0_p, dst0_p, zeros1)

    out = pl.pallas_call(
        _fin_body,
        grid=(grid,),
        in_specs=[
            pl.BlockSpec((NC, BLK, D1), lambda i: (0, i, 0)),
            pl.BlockSpec((BLK, 1), lambda i: (i, 0)),
            pl.BlockSpec((BLK, N_CLASSES), lambda i: (i, 0)),
        ],
        out_specs=pl.BlockSpec((BLK, N_CLASSES), lambda i: (i, 0)),
        out_shape=jax.ShapeDtypeStruct((N_NODES, N_CLASSES), jnp.float32),
    )(acc1, dinv, z1)

    return out


# speedup vs baseline: 1.0230x; 1.0230x over previous
import functools

import jax
import jax.numpy as jnp
from jax import lax
from jax.experimental import pallas as pl
from jax.experimental.pallas import tpu as pltpu
from jax.experimental.pallas import tpu_sc as plsc

N_NODES = 10000
N_EDGES = 320000
D_FEAT = 128
N_CLASSES = 47

NC = 2
NS = 16
NW = NC * NS
CHUNK = 128
NCH = 80
EPAD = NW * NCH * CHUNK
D0 = 144
D1 = 48
NPAD = 10016
ROWS_PER_TILE = NPAD // NS

BLK = 1000


def _make_sc_scatter(d, nbuf, g, stage_table=False):
    mesh = plsc.VectorSubcoreMesh(core_axis_name="c", subcore_axis_name="s")
    scratch = [
        pltpu.VMEM((g, CHUNK), jnp.int32),
        pltpu.VMEM((g, CHUNK), jnp.int32),
        pltpu.VMEM((nbuf, CHUNK, d), jnp.float32),
        pltpu.VMEM_SHARED((NPAD, d), jnp.float32),
        [pltpu.SemaphoreType.DMA] * nbuf,
    ]
    if stage_table:
        scratch.append(pltpu.VMEM_SHARED((N_NODES, d), jnp.float32))

    @functools.partial(
        pl.kernel,
        out_type=jax.ShapeDtypeStruct((NC, NPAD, d), jnp.float32),
        mesh=mesh,
        scratch_types=scratch,
        compiler_params=pltpu.CompilerParams(use_tc_tiling_on_sc=False),
    )
    def sc_scatter(y_hbm, src_hbm, dst_hbm, z_hbm, out_hbm,
                   srcv, dstv, rows, acc, sems, *maybe_tab):
        cid = lax.axis_index("c")
        sid = lax.axis_index("s")
        r0 = sid * ROWS_PER_TILE
        pltpu.sync_copy(z_hbm.at[pl.ds(r0, ROWS_PER_TILE)],
                        acc.at[pl.ds(r0, ROWS_PER_TILE)])
        if stage_table:
            tab = maybe_tab[0]
            tr = N_NODES // NS
            pltpu.sync_copy(y_hbm.at[pl.ds(sid * tr, tr)],
                            tab.at[pl.ds(sid * tr, tr)])
            src_tab = tab
        else:
            src_tab = y_hbm
        plsc.subcore_barrier()

        def group(gi, carry):
            pltpu.sync_copy(src_hbm.at[sid, pl.ds(cid * NCH + gi * g, g)], srcv)
            pltpu.sync_copy(dst_hbm.at[sid, pl.ds(cid * NCH + gi * g, g)], dstv)
            for b in range(nbuf):
                pltpu.async_copy(src_tab.at[srcv.at[b]], rows.at[b], sems[b])

            def body(t, carry2):
                for b in range(nbuf):
                    j = t * nbuf + b
                    pltpu.make_async_copy(
                        src_tab.at[srcv.at[j]], rows.at[b], sems[b]).wait()
                    pltpu.sync_copy(rows.at[b], acc.at[dstv.at[j]], add=True)

                    @pl.when(j + nbuf < g)
                    def _():
                        pltpu.async_copy(
                            src_tab.at[srcv.at[j + nbuf]], rows.at[b], sems[b])
                return carry2

            return lax.fori_loop(0, g // nbuf, body, carry)

        lax.fori_loop(0, NCH // g, group, 0)
        plsc.subcore_barrier()
        pltpu.sync_copy(acc.at[pl.ds(r0, ROWS_PER_TILE)],
                        out_hbm.at[cid, pl.ds(r0, ROWS_PER_TILE)])

    return sc_scatter


_sc_scatter_d1 = _make_sc_scatter(D1, nbuf=4, g=80, stage_table=True)

DSP = 64
DDEG = 16
NCH2 = 160


def _make_sc_scatter_split(nbuf, g):
    mesh = plsc.VectorSubcoreMesh(core_axis_name="c", subcore_axis_name="s")

    @functools.partial(
        pl.kernel,
        out_type=(jax.ShapeDtypeStruct((NC, NPAD, DSP), jnp.float32),
                  jax.ShapeDtypeStruct((NC, NPAD, DDEG), jnp.float32)),
        mesh=mesh,
        scratch_types=[
            pltpu.VMEM((g, CHUNK), jnp.int32),
            pltpu.VMEM((g, CHUNK), jnp.int32),
            pltpu.VMEM((nbuf, CHUNK, DSP), jnp.float32),
            pltpu.VMEM((CHUNK, DDEG), jnp.float32),
            pltpu.VMEM_SHARED((NPAD, DSP), jnp.float32),
            pltpu.VMEM_SHARED((NPAD, DDEG), jnp.float32),
            pltpu.VMEM_SHARED((N_NODES, DSP), jnp.float32),
            [pltpu.SemaphoreType.DMA] * nbuf,
        ],
        compiler_params=pltpu.CompilerParams(use_tc_tiling_on_sc=False),
    )
    def sc_scatter(ya_hbm, yb_hbm, src_hbm, dst_hbm, z_hbm, zd_hbm, ones_hbm,
                   out_hbm, deg_hbm, srcv, dstv, rows, onesv, acc, dacc, tab,
                   sems):
        cid = lax.axis_index("c")
        sid = lax.axis_index("s")
        r0 = sid * ROWS_PER_TILE
        pltpu.sync_copy(z_hbm.at[pl.ds(r0, ROWS_PER_TILE)],
                        acc.at[pl.ds(r0, ROWS_PER_TILE)])
        pltpu.sync_copy(zd_hbm.at[pl.ds(r0, ROWS_PER_TILE)],
                        dacc.at[pl.ds(r0, ROWS_PER_TILE)])
        pltpu.sync_copy(ones_hbm, onesv)
        tr = N_NODES // NS

        @pl.when(cid == 0)
        def _():
            pltpu.sync_copy(ya_hbm.at[pl.ds(sid * tr, tr)],
                            tab.at[pl.ds(sid * tr, tr)])

        @pl.when(cid == 1)
        def _():
            pltpu.sync_copy(yb_hbm.at[pl.ds(sid * tr, tr)],
                            tab.at[pl.ds(sid * tr, tr)])

        plsc.subcore_barrier()

        def group(gi, carry):
            pltpu.sync_copy(src_hbm.at[sid, pl.ds(gi * g, g)], srcv)
            pltpu.sync_copy(dst_hbm.at[sid, pl.ds(gi * g, g)], dstv)
            for b in range(nbuf):
                pltpu.async_copy(tab.at[srcv.at[b]], rows.at[b], sems[b])

            @pl.when(cid == 0)
            def _():
                for jj in range(g // 2):
                    pltpu.sync_copy(onesv, dacc.at[dstv.at[jj]], add=True)

            @pl.when(cid == 1)
            def _():
                for jj in range(g // 2, g):
                    pltpu.sync_copy(onesv, dacc.at[dstv.at[jj]], add=True)

            def body(t, carry2):
                for b in range(nbuf):
                    j = t * nbuf + b
                    pltpu.make_async_copy(
                        tab.at[srcv.at[j]], rows.at[b], sems[b]).wait()
                    pltpu.sync_copy(rows.at[b], acc.at[dstv.at[j]], add=True)

                    @pl.when(j + nbuf < g)
                    def _():
                        pltpu.async_copy(
                            tab.at[srcv.at[j + nbuf]], rows.at[b], sems[b])
                return carry2

            return lax.fori_loop(0, g // nbuf, body, carry)

        lax.fori_loop(0, NCH2 // g, group, 0)
        plsc.subcore_barrier()
        pltpu.sync_copy(acc.at[pl.ds(r0, ROWS_PER_TILE)],
                        out_hbm.at[cid, pl.ds(r0, ROWS_PER_TILE)])
        pltpu.sync_copy(dacc.at[pl.ds(r0, ROWS_PER_TILE)],
                        deg_hbm.at[cid, pl.ds(r0, ROWS_PER_TILE)])

    return sc_scatter


_sc_scatter_l0 = _make_sc_scatter_split(nbuf=4, g=8)


def _mm0_body(x_ref, w0a_ref, w0b_ref, ws_ref, b0_ref,
              ya_ref, yb_ref, s0_ref):
    xb = x_ref[...]
    ya_ref[...] = jnp.dot(xb, w0a_ref[...], preferred_element_type=jnp.float32)
    yb_ref[...] = jnp.dot(xb, w0b_ref[...], preferred_element_type=jnp.float32)
    s0_ref[...] = jnp.dot(xb, ws_ref[...],
                          preferred_element_type=jnp.float32) + b0_ref[...]


def _mid_body(acc_ref, deg_ref, s0_ref, w1_ref, ws1_ref, b1_ref,
              y1_ref, z1_ref, dinv_ref):
    agg = jnp.concatenate([acc_ref[0], acc_ref[1]], axis=1)
    deg = deg_ref[0][:, 0:1] + deg_ref[1][:, 0:1]
    dinv = 1.0 / jnp.maximum(deg, 1.0)
    h = jnp.maximum(agg * dinv + s0_ref[...], 0.0)
    y1_ref[...] = jnp.dot(h, w1_ref[...], preferred_element_type=jnp.float32)
    z1_ref[...] = jnp.dot(h, ws1_ref[...],
                          preferred_element_type=jnp.float32) + b1_ref[...]
    dinv_ref[...] = dinv


def _fin_body(acc_ref, dinv_ref, z1_ref, out_ref):
    a = acc_ref[0] + acc_ref[1]
    out_ref[...] = a[:, :N_CLASSES] * dinv_ref[...] + z1_ref[...]


def kernel(x, edge_index, W_neigh_0, W_self_0, b_0, W_neigh_1, W_self_1, b_1):
    src = edge_index[0].astype(jnp.int32)
    dst = edge_index[1].astype(jnp.int32)
    epw0 = N_EDGES // NS
    ppw0 = NCH2 * CHUNK - epw0
    pad_src0 = jnp.zeros((NS, ppw0), jnp.int32)
    pad_dst0 = jnp.broadcast_to(
        N_NODES + (jnp.arange(ppw0, dtype=jnp.int32) % (NPAD - N_NODES)),
        (NS, ppw0))
    src0_p = jnp.concatenate(
        [src.reshape(NS, epw0), pad_src0], axis=1).reshape(NS, NCH2, CHUNK)
    dst0_p = jnp.concatenate(
        [dst.reshape(NS, epw0), pad_dst0], axis=1).reshape(NS, NCH2, CHUNK)
    zeros_sp = jnp.zeros((NPAD, DSP), jnp.float32)
    zeros1 = jnp.zeros((NPAD, D1), jnp.float32)

    w0a = W_neigh_0[:, :DSP]
    w0b = W_neigh_0[:, DSP:]
    w1p = jnp.pad(W_neigh_1, ((0, 0), (0, D1 - N_CLASSES)))

    grid = N_NODES // BLK
    y0a, y0b, s0 = pl.pallas_call(
        _mm0_body,
        grid=(grid,),
        in_specs=[
            pl.BlockSpec((BLK, D_FEAT), lambda i: (i, 0)),
            pl.BlockSpec((D_FEAT, DSP), lambda i: (0, 0)),
            pl.BlockSpec((D_FEAT, DSP), lambda i: (0, 0)),
            pl.BlockSpec((D_FEAT, D_FEAT), lambda i: (0, 0)),
            pl.BlockSpec((1, D_FEAT), lambda i: (0, 0)),
        ],
        out_specs=[
            pl.BlockSpec((BLK, DSP), lambda i: (i, 0)),
            pl.BlockSpec((BLK, DSP), lambda i: (i, 0)),
            pl.BlockSpec((BLK, D_FEAT), lambda i: (i, 0)),
        ],
        out_shape=[
            jax.ShapeDtypeStruct((N_NODES, DSP), jnp.float32),
            jax.ShapeDtypeStruct((N_NODES, DSP), jnp.float32),
            jax.ShapeDtypeStruct((N_NODES, D_FEAT), jnp.float32),
        ],
    )(x, w0a, w0b, W_self_0, b_0[None, :])

    zeros_d = jnp.zeros((NPAD, DDEG), jnp.float32)
    ones_r = jnp.ones((CHUNK, DDEG), jnp.float32)
    acc0, degs = _sc_scatter_l0(y0a, y0b, src0_p, dst0_p, zeros_sp,
                                zeros_d, ones_r)

    y1p, z1, dinv = pl.pallas_call(
        _mid_body,
        grid=(grid,),
        in_specs=[
            pl.BlockSpec((NC, BLK, DSP), lambda i: (0, i, 0)),
            pl.BlockSpec((NC, BLK, DDEG), lambda i: (0, i, 0)),
            pl.BlockSpec((BLK, D_FEAT), lambda i: (i, 0)),
            pl.BlockSpec((D_FEAT, D1), lambda i: (0, 0)),
            pl.BlockSpec((D_FEAT, N_CLASSES), lambda i: (0, 0)),
            pl.BlockSpec((1, N_CLASSES), lambda i: (0, 0)),
        ],
        out_specs=[
            pl.BlockSpec((BLK, D1), lambda i: (i, 0)),
            pl.BlockSpec((BLK, N_CLASSES), lambda i: (i, 0)),
            pl.BlockSpec((BLK, 1), lambda i: (i, 0)),
        ],
        out_shape=[
            jax.ShapeDtypeStruct((N_NODES, D1), jnp.float32),
            jax.ShapeDtypeStruct((N_NODES, N_CLASSES), jnp.float32),
            jax.ShapeDtypeStruct((N_NODES, 1), jnp.float32),
        ],
    )(acc0, degs, s0, w1p, W_self_1, b_1[None, :])

    acc1 = _sc_scatter_d1(y1p, src0_p, dst0_p, zeros1)

    out = pl.pallas_call(
        _fin_body,
        grid=(grid,),
        in_specs=[
            pl.BlockSpec((NC, BLK, D1), lambda i: (0, i, 0)),
            pl.BlockSpec((BLK, 1), lambda i: (i, 0)),
            pl.BlockSpec((BLK, N_CLASSES), lambda i: (i, 0)),
        ],
        out_specs=pl.BlockSpec((BLK, N_CLASSES), lambda i: (i, 0)),
        out_shape=jax.ShapeDtypeStruct((N_NODES, N_CLASSES), jnp.float32),
    )(acc1, dinv, z1)

    return out
